# baseline (speedup 1.0000x reference)
import functools

import jax
import jax.numpy as jnp
from jax import lax
from jax.experimental import pallas as pl
from jax.experimental.pallas import tpu as pltpu
from jax.experimental.pallas import tpu_sc as plsc

BATCH = 16384
WIDTH = 64
BINS = 1000000
LANES = 16

_Q = (BINS + 127) // 128
_info = plsc.get_sparse_core_info()
_NC = _info.num_cores
_NW = _info.num_cores * _info.num_subcores
_QPW = (_Q + _NW - 1) // _NW
_BPW = BATCH // _NW
_G = 16
_NBUF = 7

_mesh = plsc.VectorSubcoreMesh(core_axis_name="c", subcore_axis_name="s")
_params = pltpu.CompilerParams(
    use_tc_tiling_on_sc=True,
    needs_layout_passes=False,
    disable_bounds_checks=True,
)


@functools.partial(
    pl.kernel,
    mesh=_mesh,
    compiler_params=_params,
    out_type=jax.ShapeDtypeStruct((BATCH, 128), jnp.float32),
    scratch_types=[
        pltpu.VMEM((BATCH,), jnp.int32),
        pltpu.VMEM((BATCH + LANES,), jnp.int32),
        pltpu.VMEM((BATCH + LANES,), jnp.int32),
        pltpu.VMEM((BATCH + LANES,), jnp.int32),
        pltpu.SMEM((_QPW + 1,), jnp.int32),
        pltpu.SMEM((_QPW + 1,), jnp.int32),
        pltpu.SMEM((_QPW + 1,), jnp.int32),
        pltpu.VMEM((_NBUF, WIDTH, 128), jnp.float32),
        pltpu.VMEM((_G, 128), jnp.float32),
        pltpu.VMEM((_G,), jnp.int32),
        pltpu.SemaphoreType.DMA((_NBUF,)),
        pltpu.SemaphoreType.DMA,
    ],
)
def _extract_kernel(idx_hbm, tableT_hbm, scratch_hbm, idx_v, hq_v, hp_v, sp_v,
                    cnt_s, off_s, db_s, slab_v, rows_v, dst_v, sems, sem2):
    wid = lax.axis_index("s") * _NC + lax.axis_index("c")
    qlo = wid * _QPW

    pltpu.sync_copy(idx_hbm, idx_v)
    lane_iota = lax.iota(jnp.int32, LANES)

    def _sload(ref, k):
        return ref[pl.ds(k, LANES)][0]

    def _splat(val):
        return jax.lax.broadcast(val, (LANES,)).astype(jnp.int32)

    def _sstore(ref, p, val):
        plsc.store_scatter(ref, [_splat(p)], _splat(val), mask=lane_iota == 0)

    def scan_step(t4, nhits):
        qrels = []
        masks = []
        csums = []
        for u in range(4):
            t = t4 * 4 + u
            v = idx_v[pl.ds(t * LANES, LANES)]
            qrel = lax.shift_right_logical(v, 7) - qlo
            m = qrel.astype(jnp.uint32) < jnp.uint32(_QPW)
            qrels.append(qrel)
            masks.append(m)
            csums.append(plsc.cumsum(m.astype(jnp.int32)))
        for u in range(4):
            t = t4 * 4 + u
            v = idx_v[pl.ds(t * LANES, LANES)]
            pos = nhits + csums[u] - 1
            plsc.store_scatter(hq_v, [pos], qrels[u], mask=masks[u])
            packed = (lane_iota + t * LANES) * 128 + (v & 127)
            plsc.store_scatter(hp_v, [pos], packed, mask=masks[u])
            nhits = nhits + csums[u][LANES - 1]
        return nhits

    nhits = lax.fori_loop(0, BATCH // LANES // 4, scan_step, jnp.int32(0))
    plsc.store_scatter(
        hq_v, [nhits + lane_iota], jax.lax.broadcast(jnp.int32(_QPW), (LANES,))
    )

    def zero_step(b, c):
        cnt_s[b] = 0
        return c

    lax.fori_loop(0, _QPW + 1, zero_step, 0)

    nblk = lax.shift_right_logical(nhits + LANES - 1, 4)

    def count_step(t, c):
        qv = hq_v[pl.ds(t * LANES, LANES)]
        for j in range(LANES):
            q = qv[j]
            cnt_s[q] = cnt_s[q] + 1
        return c

    lax.fori_loop(0, nblk, count_step, 0)

    def prefix_step(b, carry):
        acc, nd = carry
        off_s[b] = acc
        c = cnt_s[b]

        @pl.when((c > 0) & (b < _QPW))
        def _d():
            db_s[nd] = b

        return (acc + c, nd + jnp.where((c > 0) & (b < _QPW), 1, 0))

    _, ndq = lax.fori_loop(0, _QPW + 1, prefix_step, (jnp.int32(0), jnp.int32(0)))

    def place_step(t, c):
        qv = hq_v[pl.ds(t * LANES, LANES)]
        pv = hp_v[pl.ds(t * LANES, LANES)]
        for j in range(LANES):
            q = qv[j]
            p = off_s[q]
            off_s[q] = p + 1
            _sstore(sp_v, p, pv[j])
        return c

    lax.fori_loop(0, nblk, place_step, 0)

    def issue(d):
        b = db_s[d]
        sstart = (qlo + b) * 128
        pltpu.async_copy(
            tableT_hbm.at[:, pl.ds(pl.multiple_of(sstart, 128), 128)],
            slab_v.at[lax.rem(d, _NBUF)],
            sems.at[lax.rem(d, _NBUF)],
        )

    def prime_step(d, c):
        @pl.when(d < ndq)
        def _p():
            issue(d)
        return c

    lax.fori_loop(0, _NBUF - 1, prime_step, 0)

    def write_row(r, j, vals):
        rows_v[r, pl.ds(LANES * j, LANES)] = vals

    def flush(nrows, fill_i):
        def pad_step(r, c):
            @pl.when(r >= nrows)
            def _w():
                _sstore(dst_v, r, fill_i)
                for j in range(128 // LANES):
                    vals = plsc.load_gather(
                        rows_v, [_splat(nrows - 1), lane_iota + LANES * j]
                    )
                    write_row(r, j, vals)
            return c

        lax.fori_loop(0, _G, pad_step, 0)
        pltpu.async_copy(rows_v, scratch_hbm.at[dst_v], sem2).wait()

    def dbin_step(d, carry):
        nrows, fill_i = carry
        sel = lax.rem(d, _NBUF)
        b = db_s[d]
        pltpu.make_async_copy(
            tableT_hbm.at[:, pl.ds(pl.multiple_of(0, 128), 128)],
            slab_v.at[sel],
            sems.at[sel],
        ).wait()

        @pl.when(d + _NBUF - 1 < ndq)
        def _n():
            issue(d + _NBUF - 1)

        start = jnp.where(b == 0, 0, off_s[jnp.maximum(b - 1, 0)])
        end = off_s[b]

        def hit_step(k, carry2):
            nrows2, _fi = carry2
            p = _sload(sp_v, k)
            i = lax.shift_right_logical(p, 7)
            l = p & 127
            for j in range(WIDTH // LANES):
                vals = plsc.load_gather(
                    slab_v.at[sel], [lane_iota + LANES * j, _splat(l)]
                )
                write_row(nrows2, j, vals)
            _sstore(dst_v, nrows2, i)

            @pl.when(nrows2 == _G - 1)
            def _f():
                flush(jnp.int32(_G), i)

            return (lax.rem(nrows2 + 1, _G), i)

        return lax.fori_loop(start, end, hit_step, (nrows, fill_i))

    nrows, fill_i = lax.fori_loop(0, ndq, dbin_step, (jnp.int32(0), jnp.int32(0)))

    @pl.when(nrows > 0)
    def _tail():
        flush(nrows, fill_i)


def kernel(y, genre_emb):
    scratch = _extract_kernel(y.astype(jnp.int32), genre_emb.T)
    return scratch[:, :WIDTH][:, None, :]
